# baseline (speedup 1.0000x reference)
import functools

import jax
import jax.numpy as jnp
from jax import lax
from jax.experimental import pallas as pl
from jax.experimental.pallas import tpu as pltpu
from jax.experimental.pallas import tpu_sc as plsc

N_U = 2048
N_I = 2048
D = 64
EPS = 1e-7

N_EDGE = 32768
N_TILES = 16
EPT = N_EDGE // N_TILES
CHUNK = (N_U // 8) * N_I
ZBLK = CHUNK // N_TILES
LANES = 16
N_PASS = N_U * N_I // CHUNK // 2


def _sc_build_body(rows_h, cols_h, zeros_h, out_h,
                   r_v, c_v, flat_v, idx_v, gidx_v, val_v, z_v, acc):
    cid = lax.axis_index("c")
    sid = lax.axis_index("s")

    base = sid * EPT
    pltpu.sync_copy(rows_h.at[pl.ds(base, EPT)], r_v)
    pltpu.sync_copy(cols_h.at[pl.ds(base, EPT)], c_v)
    pltpu.sync_copy(zeros_h, z_v)

    def flat_body(i, _):
        rr = r_v[pl.ds(i * LANES, LANES)]
        cc = c_v[pl.ds(i * LANES, LANES)]
        flat_v[pl.ds(i * LANES, LANES)] = rr * N_I + cc
        return 0

    lax.fori_loop(0, EPT // LANES, flat_body, 0)

    half = cid * (N_U * N_I // 2)
    for j in range(N_PASS):
        pltpu.sync_copy(z_v.at[pl.ds(0, ZBLK)],
                        out_h.at[pl.ds(half + (sid * N_PASS + j) * ZBLK,
                                       ZBLK)])
    plsc.subcore_barrier()

    for p in range(N_PASS):
        chunk_base = (cid * N_PASS + p) * CHUNK

        def idx_body(i, _):
            fl = flat_v[pl.ds(i * LANES, LANES)]
            loc = fl - chunk_base
            valid = (loc >= 0) & (loc < CHUNK)
            idx_v[pl.ds(i * LANES, LANES)] = jnp.where(valid, loc, CHUNK)
            gidx_v[pl.ds(i * LANES, LANES)] = jnp.where(valid, fl, N_U * N_I)
            return 0

        lax.fori_loop(0, EPT // LANES, idx_body, 0)
        pltpu.sync_copy(z_v.at[pl.ds(0, EPT)], acc.at[idx_v])
        plsc.subcore_barrier()
        pltpu.sync_copy(z_v.at[pl.ds(ZBLK, EPT)], acc.at[idx_v], add=True)
        plsc.subcore_barrier()
        pltpu.sync_copy(acc.at[idx_v], val_v)
        pltpu.sync_copy(val_v, out_h.at[gidx_v])
        plsc.subcore_barrier()


@jax.jit
def _sc_build(rows, cols):
    zo = jnp.zeros((ZBLK + EPT,), jnp.float32).at[ZBLK:].set(1.0)
    out = jax.ShapeDtypeStruct((N_U * N_I + LANES,), jnp.float32)
    f = pl.kernel(
        _sc_build_body,
        out_type=out,
        mesh=plsc.VectorSubcoreMesh(core_axis_name="c", subcore_axis_name="s",
                                    num_cores=2, num_subcores=16),
        scratch_types=[
            pltpu.VMEM((EPT,), jnp.int32),
            pltpu.VMEM((EPT,), jnp.int32),
            pltpu.VMEM((EPT,), jnp.int32),
            pltpu.VMEM((EPT,), jnp.int32),
            pltpu.VMEM((EPT,), jnp.int32),
            pltpu.VMEM((EPT,), jnp.float32),
            pltpu.VMEM((ZBLK + EPT,), jnp.float32),
            pltpu.VMEM_SHARED((CHUNK + LANES,), jnp.float32),
        ],
    )
    return f(rows, cols, zo)[:N_U * N_I]


def _mm(A, B):
    return jax.lax.dot_general(A, B.astype(A.dtype), (((1,), (0,)), ((), ())),
                               preferred_element_type=jnp.float32)


def _mmT(A, B):
    return jax.lax.dot_general(A, B.astype(A.dtype), (((0,), (0,)), ((), ())),
                               preferred_element_type=jnp.float32)


def _dhcf_body(H_ref, u_ref, i_ref, W0_ref, b0_ref, W1_ref, b1_ref,
               u1_ref, u2_ref, i1_ref, i2_ref):
    H = H_ref[...]

    rs = jnp.sum(H.astype(jnp.float32), axis=1, keepdims=True)
    ones_c = jnp.ones((N_U, 1), jnp.float32)
    q = _mmT(H, jnp.concatenate([rs, ones_c], axis=1))
    p0 = q[:, 0:1]
    cs = q[:, 1:2]
    gq = _mm(H, q)
    Grs = gq[:, 0:1]
    Gcs = _mmT(H, gq[:, 1:2])

    dv_u = jax.lax.rsqrt(rs + Grs + EPS)
    de1_u = 1.0 / (cs + EPS)
    de2_u = 1.0 / (Gcs + EPS)
    dv_i = jax.lax.rsqrt(cs + Gcs + EPS)
    de1_i = 1.0 / (rs + EPS)
    de2_i = 1.0 / (Grs + EPS)

    U = u_ref[...]
    I = i_ref[...]
    W0 = W0_ref[...]
    b0 = b0_ref[...]
    W1 = W1_ref[...]
    b1 = b1_ref[...]

    v1 = _mmT(H, dv_u * U)
    r = _mm(H, jnp.concatenate([v1, dv_i * I], axis=1))
    v2, w1 = r[:, :D], r[:, D:]
    r = _mmT(H, jnp.concatenate([v2, w1], axis=1))
    v3, w2 = r[:, :D], r[:, D:]
    r = _mm(H, jnp.concatenate([de2_u * v3, w2], axis=1))
    v4, w3 = r[:, :D], r[:, D:]
    r = _mmT(H, jnp.concatenate([v4, de2_i * w3], axis=1))
    v5, w4 = r[:, :D], r[:, D:]
    r = _mm(H, jnp.concatenate([de1_u * v1 + v5, w4], axis=1))
    v6, w5 = r[:, :D], r[:, D:]
    U1 = _mm(dv_u * v6 + U, W0) + b0
    u1_ref[...] = U1
    r = _mmT(H, jnp.concatenate([dv_u * U1, de1_i * w1 + w5], axis=1))
    a2, w6 = r[:, :D], r[:, D:]
    I1 = _mm(dv_i * w6 + I, W0) + b0
    i1_ref[...] = I1
    r = _mm(H, jnp.concatenate([a2, dv_i * I1], axis=1))
    v2b, w1b = r[:, :D], r[:, D:]
    r = _mmT(H, jnp.concatenate([v2b, w1b], axis=1))
    v3b, w2b = r[:, :D], r[:, D:]
    r = _mm(H, jnp.concatenate([de2_u * v3b, w2b], axis=1))
    v4b, w3b = r[:, :D], r[:, D:]
    r = _mmT(H, jnp.concatenate([v4b, de2_i * w3b], axis=1))
    v5b, w4b = r[:, :D], r[:, D:]
    r = _mm(H, jnp.concatenate([de1_u * a2 + v5b, w4b], axis=1))
    v6b, w5b = r[:, :D], r[:, D:]
    u2_ref[...] = _mm(dv_u * v6b + U1, W1) + b1
    w6b = _mmT(H, de1_i * w1b + w5b)
    i2_ref[...] = _mm(dv_i * w6b + I1, W1) + b1


@functools.partial(jax.jit, static_argnames=("interpret",))
def _dhcf_tc(H, user_emb, item_emb, W0, b0, W1, b1, interpret=False):
    out = jax.ShapeDtypeStruct((N_U, D), jnp.float32)
    return pl.pallas_call(
        _dhcf_body,
        out_shape=(out, out, out, out),
        interpret=interpret,
    )(H, user_emb, item_emb, W0, b0.reshape(1, D), W1, b1.reshape(1, D))


def kernel(user_emb, item_emb, W0, b0, W1, b1, rows, cols):
    H = _sc_build(rows, cols).reshape(N_U, N_I).astype(jnp.bfloat16)
    u1, u2, i1, i2 = _dhcf_tc(H, user_emb, item_emb, W0, b0, W1, b1)
    U_out = jnp.concatenate([user_emb, u1, u2], axis=1)
    I_out = jnp.concatenate([item_emb, i1, i2], axis=1)
    return (U_out, I_out)


# speedup vs baseline: 138.0272x; 138.0272x over previous
import functools

import jax
import jax.numpy as jnp
from jax import lax
from jax.experimental import pallas as pl
from jax.experimental.pallas import tpu as pltpu
from jax.experimental.pallas import tpu_sc as plsc

N_U = 2048
N_I = 2048
D = 64
EPS = 1e-7

N_EDGE = 32768
N_TILES = 16
EPT = N_EDGE // N_TILES
CHUNK = (N_U // 8) * N_I
ZBLK = CHUNK // N_TILES
LANES = 16
N_PASS = N_U * N_I // CHUNK // 2


def _sc_build_body(rows_h, cols_h, zeros_h, out_h,
                   r_v, c_v, flat_v, idx_v, gidx_v, val_v, keep_v, z_v, acc):
    cid = lax.axis_index("c")
    sid = lax.axis_index("s")

    base = sid * EPT
    pltpu.sync_copy(rows_h.at[pl.ds(base, EPT)], r_v)
    pltpu.sync_copy(cols_h.at[pl.ds(base, EPT)], c_v)
    pltpu.sync_copy(zeros_h, z_v)

    def flat_body(i, _):
        rr = r_v[pl.ds(i * LANES, LANES)]
        cc = c_v[pl.ds(i * LANES, LANES)]
        flat_v[pl.ds(i * LANES, LANES)] = rr * N_I + cc
        return 0

    lax.fori_loop(0, EPT // LANES, flat_body, 0)

    half = cid * (N_U * N_I // 2)
    for j in range(N_PASS):
        pltpu.sync_copy(z_v.at[pl.ds(0, ZBLK)],
                        out_h.at[pl.ds(half + (sid * N_PASS + j) * ZBLK,
                                       ZBLK)])
    plsc.subcore_barrier()

    for p in range(N_PASS):
        chunk_base = (cid * N_PASS + p) * CHUNK

        def idx_body(i, _):
            fl = flat_v[pl.ds(i * LANES, LANES)]
            loc = fl - chunk_base
            valid = (loc >= 0) & (loc < CHUNK)
            trash = CHUNK + sid * EPT + i * LANES + lax.iota(jnp.int32, 16)
            idx_v[pl.ds(i * LANES, LANES)] = jnp.where(valid, loc, trash)
            return 0

        lax.fori_loop(0, EPT // LANES, idx_body, 0)
        pltpu.sync_copy(z_v.at[pl.ds(0, EPT)], acc.at[idx_v])
        plsc.subcore_barrier()
        pltpu.sync_copy(z_v.at[pl.ds(ZBLK, EPT)], acc.at[idx_v], add=True)
        plsc.subcore_barrier()
        pltpu.sync_copy(acc.at[idx_v], val_v)

        def merge_body(i, _):
            fl = flat_v[pl.ds(i * LANES, LANES)]
            loc = fl - chunk_base
            valid = (loc >= 0) & (loc < CHUNK)
            kept = jnp.where(valid, val_v[pl.ds(i * LANES, LANES)],
                             keep_v[pl.ds(i * LANES, LANES)])
            keep_v[pl.ds(i * LANES, LANES)] = kept
            return 0

        lax.fori_loop(0, EPT // LANES, merge_body, 0)
        plsc.subcore_barrier()

    half_lo = cid * (N_U * N_I // 2)

    def gidx_body(i, _):
        fl = flat_v[pl.ds(i * LANES, LANES)]
        mine = (fl >= half_lo) & (fl < half_lo + N_U * N_I // 2)
        trash = (N_U * N_I + (cid * N_TILES + sid) * EPT + i * LANES
                 + lax.iota(jnp.int32, 16))
        gidx_v[pl.ds(i * LANES, LANES)] = jnp.where(mine, fl, trash)
        return 0

    lax.fori_loop(0, EPT // LANES, gidx_body, 0)
    pltpu.sync_copy(keep_v, out_h.at[gidx_v])


@jax.jit
def _sc_build(rows, cols):
    zo = jnp.zeros((ZBLK + EPT,), jnp.float32).at[ZBLK:].set(1.0)
    out = jax.ShapeDtypeStruct((N_U * N_I + 2 * N_TILES * EPT,), jnp.float32)
    f = pl.kernel(
        _sc_build_body,
        out_type=out,
        mesh=plsc.VectorSubcoreMesh(core_axis_name="c", subcore_axis_name="s",
                                    num_cores=2, num_subcores=16),
        scratch_types=[
            pltpu.VMEM((EPT,), jnp.int32),
            pltpu.VMEM((EPT,), jnp.int32),
            pltpu.VMEM((EPT,), jnp.int32),
            pltpu.VMEM((EPT,), jnp.int32),
            pltpu.VMEM((EPT,), jnp.int32),
            pltpu.VMEM((EPT,), jnp.float32),
            pltpu.VMEM((EPT,), jnp.float32),
            pltpu.VMEM((ZBLK + EPT,), jnp.float32),
            pltpu.VMEM_SHARED((CHUNK + N_TILES * EPT,), jnp.float32),
        ],
    )
    return f(rows, cols, zo)[:N_U * N_I]


def _mm(A, B):
    return jax.lax.dot_general(A, B.astype(A.dtype), (((1,), (0,)), ((), ())),
                               preferred_element_type=jnp.float32)


def _mmT(A, B):
    return jax.lax.dot_general(A, B.astype(A.dtype), (((0,), (0,)), ((), ())),
                               preferred_element_type=jnp.float32)


def _dhcf_body(H_ref, u_ref, i_ref, W0_ref, b0_ref, W1_ref, b1_ref,
               u1_ref, u2_ref, i1_ref, i2_ref):
    H = H_ref[...]

    rs = jnp.sum(H.astype(jnp.float32), axis=1, keepdims=True)
    ones_c = jnp.ones((N_U, 1), jnp.float32)
    q = _mmT(H, jnp.concatenate([rs, ones_c], axis=1))
    p0 = q[:, 0:1]
    cs = q[:, 1:2]
    gq = _mm(H, q)
    Grs = gq[:, 0:1]
    Gcs = _mmT(H, gq[:, 1:2])

    dv_u = jax.lax.rsqrt(rs + Grs + EPS)
    de1_u = 1.0 / (cs + EPS)
    de2_u = 1.0 / (Gcs + EPS)
    dv_i = jax.lax.rsqrt(cs + Gcs + EPS)
    de1_i = 1.0 / (rs + EPS)
    de2_i = 1.0 / (Grs + EPS)

    U = u_ref[...]
    I = i_ref[...]
    W0 = W0_ref[...]
    b0 = b0_ref[...]
    W1 = W1_ref[...]
    b1 = b1_ref[...]

    v1 = _mmT(H, dv_u * U)
    r = _mm(H, jnp.concatenate([v1, dv_i * I], axis=1))
    v2, w1 = r[:, :D], r[:, D:]
    r = _mmT(H, jnp.concatenate([v2, w1], axis=1))
    v3, w2 = r[:, :D], r[:, D:]
    r = _mm(H, jnp.concatenate([de2_u * v3, w2], axis=1))
    v4, w3 = r[:, :D], r[:, D:]
    r = _mmT(H, jnp.concatenate([v4, de2_i * w3], axis=1))
    v5, w4 = r[:, :D], r[:, D:]
    r = _mm(H, jnp.concatenate([de1_u * v1 + v5, w4], axis=1))
    v6, w5 = r[:, :D], r[:, D:]
    U1 = _mm(dv_u * v6 + U, W0) + b0
    u1_ref[...] = U1
    r = _mmT(H, jnp.concatenate([dv_u * U1, de1_i * w1 + w5], axis=1))
    a2, w6 = r[:, :D], r[:, D:]
    I1 = _mm(dv_i * w6 + I, W0) + b0
    i1_ref[...] = I1
    r = _mm(H, jnp.concatenate([a2, dv_i * I1], axis=1))
    v2b, w1b = r[:, :D], r[:, D:]
    r = _mmT(H, jnp.concatenate([v2b, w1b], axis=1))
    v3b, w2b = r[:, :D], r[:, D:]
    r = _mm(H, jnp.concatenate([de2_u * v3b, w2b], axis=1))
    v4b, w3b = r[:, :D], r[:, D:]
    r = _mmT(H, jnp.concatenate([v4b, de2_i * w3b], axis=1))
    v5b, w4b = r[:, :D], r[:, D:]
    r = _mm(H, jnp.concatenate([de1_u * a2 + v5b, w4b], axis=1))
    v6b, w5b = r[:, :D], r[:, D:]
    u2_ref[...] = _mm(dv_u * v6b + U1, W1) + b1
    w6b = _mmT(H, de1_i * w1b + w5b)
    i2_ref[...] = _mm(dv_i * w6b + I1, W1) + b1


@functools.partial(jax.jit, static_argnames=("interpret",))
def _dhcf_tc(H, user_emb, item_emb, W0, b0, W1, b1, interpret=False):
    out = jax.ShapeDtypeStruct((N_U, D), jnp.float32)
    return pl.pallas_call(
        _dhcf_body,
        out_shape=(out, out, out, out),
        interpret=interpret,
    )(H, user_emb, item_emb, W0, b0.reshape(1, D), W1, b1.reshape(1, D))


def kernel(user_emb, item_emb, W0, b0, W1, b1, rows, cols):
    H = _sc_build(rows, cols).reshape(N_U, N_I).astype(jnp.bfloat16)
    u1, u2, i1, i2 = _dhcf_tc(H, user_emb, item_emb, W0, b0, W1, b1)
    U_out = jnp.concatenate([user_emb, u1, u2], axis=1)
    I_out = jnp.concatenate([item_emb, i1, i2], axis=1)
    return (U_out, I_out)


# speedup vs baseline: 236.3441x; 1.7123x over previous
import functools

import jax
import jax.numpy as jnp
from jax import lax
from jax.experimental import pallas as pl
from jax.experimental.pallas import tpu as pltpu
from jax.experimental.pallas import tpu_sc as plsc

N_U = 2048
N_I = 2048
D = 64
EPS = 1e-7

N_EDGE = 32768
N_TILES = 16
EPT = N_EDGE // N_TILES
CHUNK = (N_U // 8) * N_I
ZBLK = CHUNK // N_TILES
LANES = 16
N_PASS = N_U * N_I // CHUNK // 2


def _sc_build_body(rows_h, cols_h, zeros_h, out_h,
                   r_v, c_v, flat_v, idx_v, ones_v, z_v, stage_v, acc):
    cid = lax.axis_index("c")
    sid = lax.axis_index("s")

    base = sid * EPT
    pltpu.sync_copy(rows_h.at[pl.ds(base, EPT)], r_v)
    pltpu.sync_copy(cols_h.at[pl.ds(base, EPT)], c_v)
    pltpu.sync_copy(zeros_h.at[pl.ds(0, ZBLK)], z_v)
    pltpu.sync_copy(zeros_h.at[pl.ds(ZBLK, EPT)], ones_v)

    def flat_body(i, _):
        rr = r_v[pl.ds(i * LANES, LANES)]
        cc = c_v[pl.ds(i * LANES, LANES)]
        flat_v[pl.ds(i * LANES, LANES)] = rr * N_I + cc
        return 0

    lax.fori_loop(0, EPT // LANES, flat_body, 0)

    for p in range(N_PASS):
        chunk_base = (cid * N_PASS + p) * CHUNK
        pltpu.sync_copy(z_v, acc.at[pl.ds(sid * ZBLK, ZBLK)])
        plsc.subcore_barrier()

        def idx_body(i, _):
            fl = flat_v[pl.ds(i * LANES, LANES)]
            loc = fl - chunk_base
            valid = (loc >= 0) & (loc < CHUNK)
            trash = CHUNK + sid * EPT + i * LANES + lax.iota(jnp.int32, 16)
            idx_v[pl.ds(i * LANES, LANES)] = jnp.where(valid, loc, trash)
            return 0

        lax.fori_loop(0, EPT // LANES, idx_body, 0)
        pltpu.sync_copy(ones_v, acc.at[idx_v], add=True)
        plsc.subcore_barrier()
        pltpu.sync_copy(acc.at[pl.ds(sid * ZBLK, ZBLK)], stage_v)
        pltpu.sync_copy(stage_v, out_h.at[pl.ds(chunk_base + sid * ZBLK,
                                                ZBLK)])
        plsc.subcore_barrier()


@jax.jit
def _sc_build(rows, cols):
    zo = jnp.zeros((ZBLK + EPT,), jnp.float32).at[ZBLK:].set(1.0)
    out = jax.ShapeDtypeStruct((N_U * N_I,), jnp.float32)
    f = pl.kernel(
        _sc_build_body,
        out_type=out,
        mesh=plsc.VectorSubcoreMesh(core_axis_name="c", subcore_axis_name="s",
                                    num_cores=2, num_subcores=16),
        scratch_types=[
            pltpu.VMEM((EPT,), jnp.int32),
            pltpu.VMEM((EPT,), jnp.int32),
            pltpu.VMEM((EPT,), jnp.int32),
            pltpu.VMEM((EPT,), jnp.int32),
            pltpu.VMEM((EPT,), jnp.float32),
            pltpu.VMEM((ZBLK,), jnp.float32),
            pltpu.VMEM((ZBLK,), jnp.float32),
            pltpu.VMEM_SHARED((CHUNK + N_TILES * EPT,), jnp.float32),
        ],
    )
    return f(rows, cols, zo)


def _mm(A, B):
    return jax.lax.dot_general(A, B.astype(A.dtype), (((1,), (0,)), ((), ())),
                               preferred_element_type=jnp.float32)


def _mmT(A, B):
    return jax.lax.dot_general(A, B.astype(A.dtype), (((0,), (0,)), ((), ())),
                               preferred_element_type=jnp.float32)


def _dhcf_body(H_ref, u_ref, i_ref, W0_ref, b0_ref, W1_ref, b1_ref,
               u1_ref, u2_ref, i1_ref, i2_ref):
    H = H_ref[...]

    rs = jnp.sum(H.astype(jnp.float32), axis=1, keepdims=True)
    ones_c = jnp.ones((N_U, 1), jnp.float32)
    q = _mmT(H, jnp.concatenate([rs, ones_c], axis=1))
    p0 = q[:, 0:1]
    cs = q[:, 1:2]
    gq = _mm(H, q)
    Grs = gq[:, 0:1]
    Gcs = _mmT(H, gq[:, 1:2])

    dv_u = jax.lax.rsqrt(rs + Grs + EPS)
    de1_u = 1.0 / (cs + EPS)
    de2_u = 1.0 / (Gcs + EPS)
    dv_i = jax.lax.rsqrt(cs + Gcs + EPS)
    de1_i = 1.0 / (rs + EPS)
    de2_i = 1.0 / (Grs + EPS)

    U = u_ref[...]
    I = i_ref[...]
    W0 = W0_ref[...]
    b0 = b0_ref[...]
    W1 = W1_ref[...]
    b1 = b1_ref[...]

    v1 = _mmT(H, dv_u * U)
    r = _mm(H, jnp.concatenate([v1, dv_i * I], axis=1))
    v2, w1 = r[:, :D], r[:, D:]
    r = _mmT(H, jnp.concatenate([v2, w1], axis=1))
    v3, w2 = r[:, :D], r[:, D:]
    r = _mm(H, jnp.concatenate([de2_u * v3, w2], axis=1))
    v4, w3 = r[:, :D], r[:, D:]
    r = _mmT(H, jnp.concatenate([v4, de2_i * w3], axis=1))
    v5, w4 = r[:, :D], r[:, D:]
    r = _mm(H, jnp.concatenate([de1_u * v1 + v5, w4], axis=1))
    v6, w5 = r[:, :D], r[:, D:]
    U1 = _mm(dv_u * v6 + U, W0) + b0
    u1_ref[...] = U1
    r = _mmT(H, jnp.concatenate([dv_u * U1, de1_i * w1 + w5], axis=1))
    a2, w6 = r[:, :D], r[:, D:]
    I1 = _mm(dv_i * w6 + I, W0) + b0
    i1_ref[...] = I1
    r = _mm(H, jnp.concatenate([a2, dv_i * I1], axis=1))
    v2b, w1b = r[:, :D], r[:, D:]
    r = _mmT(H, jnp.concatenate([v2b, w1b], axis=1))
    v3b, w2b = r[:, :D], r[:, D:]
    r = _mm(H, jnp.concatenate([de2_u * v3b, w2b], axis=1))
    v4b, w3b = r[:, :D], r[:, D:]
    r = _mmT(H, jnp.concatenate([v4b, de2_i * w3b], axis=1))
    v5b, w4b = r[:, :D], r[:, D:]
    r = _mm(H, jnp.concatenate([de1_u * a2 + v5b, w4b], axis=1))
    v6b, w5b = r[:, :D], r[:, D:]
    u2_ref[...] = _mm(dv_u * v6b + U1, W1) + b1
    w6b = _mmT(H, de1_i * w1b + w5b)
    i2_ref[...] = _mm(dv_i * w6b + I1, W1) + b1


@functools.partial(jax.jit, static_argnames=("interpret",))
def _dhcf_tc(H, user_emb, item_emb, W0, b0, W1, b1, interpret=False):
    out = jax.ShapeDtypeStruct((N_U, D), jnp.float32)
    return pl.pallas_call(
        _dhcf_body,
        out_shape=(out, out, out, out),
        interpret=interpret,
    )(H, user_emb, item_emb, W0, b0.reshape(1, D), W1, b1.reshape(1, D))


def kernel(user_emb, item_emb, W0, b0, W1, b1, rows, cols):
    H = _sc_build(rows, cols).reshape(N_U, N_I).astype(jnp.bfloat16)
    u1, u2, i1, i2 = _dhcf_tc(H, user_emb, item_emb, W0, b0, W1, b1)
    U_out = jnp.concatenate([user_emb, u1, u2], axis=1)
    I_out = jnp.concatenate([item_emb, i1, i2], axis=1)
    return (U_out, I_out)
